# baseline (speedup 1.0000x reference)
import functools

import jax
import jax.numpy as jnp
from jax import lax
from jax.experimental import pallas as pl
from jax.experimental.pallas import tpu as pltpu
from jax.experimental.pallas import tpu_sc as plsc


@functools.lru_cache(maxsize=None)
def _build(batch, datasize, class_num):
    info = plsc.get_sparse_core_info()
    nw = info.num_cores * info.num_subcores
    b_per_w = batch // nw
    assert batch % (8 * nw) == 0

    mesh = plsc.VectorSubcoreMesh(core_axis_name="c", subcore_axis_name="s")

    @functools.partial(
        pl.kernel,
        mesh=mesh,
        out_type=jax.ShapeDtypeStruct((batch, class_num), jnp.float32),
        scratch_types=[
            pltpu.VMEM((b_per_w,), jnp.int32),
            pltpu.VMEM((b_per_w, class_num), jnp.float32),
            pltpu.SemaphoreType.DMA,
        ],
        compiler_params=pltpu.CompilerParams(use_tc_tiling_on_sc=False),
    )
    def gather_kernel(idx_hbm, table_hbm, out_hbm, idx_v, rows_v, sem):
        wid = lax.axis_index("s") * info.num_cores + lax.axis_index("c")
        base = wid * b_per_w
        pltpu.sync_copy(idx_hbm.at[pl.ds(base, b_per_w)], idx_v)
        pltpu.async_copy(table_hbm.at[idx_v], rows_v, sem).wait()
        pltpu.sync_copy(rows_v, out_hbm.at[pl.ds(base, b_per_w)])

    return gather_kernel


def kernel(index, pseudo_label):
    batch = index.shape[0]
    datasize, class_num = pseudo_label.shape
    return _build(batch, datasize, class_num)(index, pseudo_label)


# speedup vs baseline: 1.0025x; 1.0025x over previous
import functools

import jax
import jax.numpy as jnp
from jax import lax
from jax.experimental import pallas as pl
from jax.experimental.pallas import tpu as pltpu
from jax.experimental.pallas import tpu_sc as plsc


@functools.lru_cache(maxsize=None)
def _build(batch, datasize, class_num):
    info = plsc.get_sparse_core_info()
    nw = info.num_cores * info.num_subcores
    b_per_w = batch // nw
    assert batch % (8 * nw) == 0

    mesh = plsc.VectorSubcoreMesh(core_axis_name="c", subcore_axis_name="s")

    nwin = 8
    w = b_per_w // nwin

    @functools.partial(
        pl.kernel,
        mesh=mesh,
        out_type=jax.ShapeDtypeStruct((batch, class_num), jnp.float32),
        scratch_types=[
            pltpu.VMEM((b_per_w,), jnp.int32),
            pltpu.VMEM((b_per_w, class_num), jnp.float32),
            pltpu.SemaphoreType.DMA((nwin,)),
            pltpu.SemaphoreType.DMA((nwin,)),
        ],
        compiler_params=pltpu.CompilerParams(use_tc_tiling_on_sc=False),
    )
    def gather_kernel(idx_hbm, table_hbm, out_hbm, idx_v, rows_v, gsem, wsem):
        wid = lax.axis_index("s") * info.num_cores + lax.axis_index("c")
        base = wid * b_per_w
        pltpu.sync_copy(idx_hbm.at[pl.ds(base, b_per_w)], idx_v)
        gathers = [
            pltpu.async_copy(
                table_hbm.at[idx_v.at[pl.ds(j * w, w)]],
                rows_v.at[pl.ds(j * w, w)],
                gsem.at[j],
            )
            for j in range(nwin)
        ]
        writes = []
        for j in range(nwin):
            gathers[j].wait()
            writes.append(
                pltpu.async_copy(
                    rows_v.at[pl.ds(j * w, w)],
                    out_hbm.at[pl.ds(base + j * w, w)],
                    wsem.at[j],
                )
            )
        for wr in writes:
            wr.wait()

    return gather_kernel


def kernel(index, pseudo_label):
    batch = index.shape[0]
    datasize, class_num = pseudo_label.shape
    return _build(batch, datasize, class_num)(index, pseudo_label)


# speedup vs baseline: 1.6605x; 1.6563x over previous
import functools

import jax
import jax.numpy as jnp
from jax import lax
from jax.experimental import pallas as pl
from jax.experimental.pallas import tpu as pltpu
from jax.experimental.pallas import tpu_sc as plsc

_L = 16


@functools.lru_cache(maxsize=None)
def _build(batch, datasize, class_num):
    info = plsc.get_sparse_core_info()
    nw = info.num_cores * info.num_subcores
    b_per_w = batch // nw

    win = 32
    nwin = b_per_w // win

    mesh = plsc.VectorSubcoreMesh(core_axis_name="c", subcore_axis_name="s")

    @functools.partial(
        pl.kernel,
        mesh=mesh,
        out_type=jax.ShapeDtypeStruct((batch, class_num), jnp.float32),
        scratch_types=[
            pltpu.VMEM((b_per_w,), jnp.int32),
            pltpu.VMEM((2, win, class_num), jnp.float32),
            pltpu.SemaphoreType.DMA,
            pltpu.SemaphoreType.DMA((nwin,)),
        ],
    )
    def gather_kernel(idx_hbm, table_hbm, out_hbm, idx_v, obuf, gsem, wsem):
        wid = lax.axis_index("s") * info.num_cores + lax.axis_index("c")
        base = wid * b_per_w
        pltpu.sync_copy(idx_hbm.at[pl.ds(base, b_per_w)], idx_v)

        writes = [None] * nwin
        for j in range(nwin):
            b = j & 1
            if j >= 2:
                writes[j - 2].wait()
            for g in range(win // _L):
                v = idx_v[pl.ds(j * win + g * _L, _L)]
                for l in range(_L):
                    i = lax.squeeze(lax.slice(v, [l], [l + 1]), [0])
                    pltpu.async_copy(
                        table_hbm.at[i], obuf.at[b, g * _L + l], gsem)
            pltpu.make_async_copy(
                table_hbm.at[pl.ds(0, win)], obuf.at[b], gsem
            ).wait()
            writes[j] = pltpu.async_copy(
                obuf.at[b],
                out_hbm.at[pl.ds(base + j * win, win)],
                wsem.at[j],
            )
        writes[nwin - 2].wait()
        writes[nwin - 1].wait()

    return gather_kernel


def kernel(index, pseudo_label):
    batch = index.shape[0]
    datasize, class_num = pseudo_label.shape
    return _build(batch, datasize, class_num)(index, pseudo_label)


# speedup vs baseline: 1.6690x; 1.0051x over previous
import functools

import jax
import jax.numpy as jnp
from jax import lax
from jax.experimental import pallas as pl
from jax.experimental.pallas import tpu as pltpu
from jax.experimental.pallas import tpu_sc as plsc

_L = 16


@functools.lru_cache(maxsize=None)
def _build(batch, datasize, class_num):
    info = plsc.get_sparse_core_info()
    nw = info.num_cores * info.num_subcores
    b_per_w = batch // nw

    win = _L
    nwin = b_per_w // win
    nbuf = 4

    mesh = plsc.VectorSubcoreMesh(core_axis_name="c", subcore_axis_name="s")

    @functools.partial(
        pl.kernel,
        mesh=mesh,
        out_type=jax.ShapeDtypeStruct((batch, class_num), jnp.float32),
        scratch_types=[
            pltpu.VMEM((b_per_w,), jnp.int32),
            pltpu.VMEM((nbuf, win, class_num), jnp.float32),
            pltpu.SemaphoreType.DMA((nbuf,)),
            pltpu.SemaphoreType.DMA((nbuf,)),
        ],
    )
    def gather_kernel(idx_hbm, table_hbm, out_hbm, idx_v, obuf, gsem, wsem):
        wid = lax.axis_index("s") * info.num_cores + lax.axis_index("c")
        base = wid * b_per_w
        pltpu.sync_copy(idx_hbm.at[pl.ds(base, b_per_w)], idx_v)

        def fire(j):
            b = j % nbuf
            v = idx_v[pl.ds(j * win, _L)]
            for l in range(_L):
                i = lax.squeeze(lax.slice(v, [l], [l + 1]), [0])
                pltpu.async_copy(table_hbm.at[i], obuf.at[b, l], gsem.at[b])

        writes = [None] * nwin
        fire(0)
        for j in range(nwin):
            b = j % nbuf
            if j + 1 < nwin:
                if j + 1 >= nbuf:
                    writes[j + 1 - nbuf].wait()
                fire(j + 1)
            pltpu.make_async_copy(
                table_hbm.at[pl.ds(0, win)], obuf.at[b], gsem.at[b]
            ).wait()
            writes[j] = pltpu.async_copy(
                obuf.at[b],
                out_hbm.at[pl.ds(base + j * win, win)],
                wsem.at[b],
            )
        for j in range(max(nwin - nbuf, 0), nwin):
            writes[j].wait()

    return gather_kernel


def kernel(index, pseudo_label):
    batch = index.shape[0]
    datasize, class_num = pseudo_label.shape
    return _build(batch, datasize, class_num)(index, pseudo_label)
